# Initial kernel scaffold; baseline (speedup 1.0000x reference)
import jax
import jax.numpy as jnp
from jax.experimental import pallas as pl


def kernel(x, ln1_g, ln1_b, in_w, in_b, out_w, out_b, ln2_g, ln2_b, gate_w, gup, down, head_w, head_b):
    raise NotImplementedError("write your pallas kernel here")


# speedup vs baseline: 1.0191x; 1.0191x over previous
import jax
import jax.numpy as jnp
from jax.experimental import pallas as pl
from jax.experimental.pallas import tpu as pltpu

H = 64
E = 4
TOPK = 2
FFN = 128
NLAYERS = 2
VOCAB = 100

Q_CHUNK = 512
K_CHUNK = 256


def _dot_t(a, b):
    return jax.lax.dot_general(
        a, b, (((1,), (1,)), ((), ())), preferred_element_type=jnp.float32)


def _dot(a, b):
    return jax.lax.dot_general(
        a, b, (((1,), (0,)), ((), ())), preferred_element_type=jnp.float32)


def _row_sum(x):
    n = x.shape[-1]
    c = min(n, 128)
    s = x[..., 0:c]
    for i in range(1, n // c):
        s = s + x[..., i * c:(i + 1) * c]
    if c > 8:
        t = s[..., 0:8]
        for i in range(1, c // 8):
            t = t + s[..., i * 8:(i + 1) * 8]
        s = t
    s = s[..., 0:4] + s[..., 4:8]
    s = s[..., 0:2] + s[..., 2:4]
    return s[..., 0:1] + s[..., 1:2]


def _layernorm(x, g, b):
    inv_n = 1.0 / x.shape[-1]
    m = _row_sum(x) * inv_n
    c = x - m
    v = _row_sum(c * c) * inv_n
    return c / jnp.sqrt(v + 1e-5) * g + b


def _fwd_kernel(x_ref, ln1_g_ref, ln1_b_ref, in_w_ref, in_b_ref, out_w_ref,
                out_b_ref, ln2_g_ref, ln2_b_ref, gate_w_ref, gup_ref,
                down_ref, head_w_ref, head_b_ref, out_ref, acc_ref):
    x = x_ref[0]
    S = x.shape[0]
    scale = 1.0 / (H ** 0.5)

    for i in range(NLAYERS):
        h = _layernorm(x, ln1_g_ref[i], ln1_b_ref[i])
        q = _dot_t(h, in_w_ref[i, :H]) + in_b_ref[i, :H]
        k = _dot_t(h, in_w_ref[i, H:2 * H]) + in_b_ref[i, H:2 * H]
        v = _dot_t(h, in_w_ref[i, 2 * H:]) + in_b_ref[i, 2 * H:]
        o_chunks = []
        for ci in range(S // Q_CHUNK):
            qb = q[ci * Q_CHUNK:(ci + 1) * Q_CHUNK]
            s = _dot_t(qb, k) * scale
            s = s - jnp.max(s, axis=-1, keepdims=True)
            p = jnp.exp(s)
            den = jnp.sum(p, axis=-1, keepdims=True)
            for kc in range(S // K_CHUNK):
                d = _dot(p[:, kc * K_CHUNK:(kc + 1) * K_CHUNK],
                         v[kc * K_CHUNK:(kc + 1) * K_CHUNK])
                if kc == 0:
                    acc_ref[...] = d
                else:
                    acc_ref[...] = acc_ref[...] + d
            o_chunks.append(acc_ref[...] / den)
        o = jnp.concatenate(o_chunks, axis=0)
        x = x + (_dot_t(o, out_w_ref[i]) + out_b_ref[i])

        h = _layernorm(x, ln2_g_ref[i], ln2_b_ref[i])
        logits = _dot_t(h, gate_w_ref[i])
        logits = logits - jnp.max(logits, axis=-1, keepdims=True)
        ex = jnp.exp(logits)
        den = (ex[:, 0:1] + ex[:, 2:3]) + (ex[:, 1:2] + ex[:, 3:4])
        probs = ex / den

        iota = jax.lax.broadcasted_iota(jnp.int32, (S, E), 1)
        m1 = jnp.max(probs, axis=-1, keepdims=True)
        i1 = jnp.min(jnp.where(probs == m1, iota, E), axis=-1, keepdims=True)
        oh1 = iota == i1
        p2 = jnp.where(oh1, -jnp.inf, probs)
        m2 = jnp.max(p2, axis=-1, keepdims=True)
        i2 = jnp.min(jnp.where(p2 == m2, iota, E), axis=-1, keepdims=True)
        oh2 = iota == i2
        w = jnp.where(oh1, m1, 0.0) + jnp.where(oh2, m2, 0.0)
        w = w / (m1 + m2)

        moe = jnp.zeros((S, H), dtype=jnp.float32)
        for e in range(E):
            he = _dot_t(h, gup_ref[i, e])
            g = he[:, :FFN]
            u = he[:, FFN:]
            a = jax.nn.silu(g) * u
            oe = _dot_t(a, down_ref[i, e])
            moe = moe + w[:, e:e + 1] * oe
        x = x + moe

    out_ref[0] = _dot_t(x, head_w_ref[:]) + head_b_ref[:]


@jax.jit
def kernel(x, ln1_g, ln1_b, in_w, in_b, out_w, out_b, ln2_g, ln2_b, gate_w,
           gup, down, head_w, head_b):
    B, S, _ = x.shape
    head_b2 = head_b.reshape(1, VOCAB)

    def whole(a):
        return pl.BlockSpec(a.shape, lambda b: (0,) * a.ndim)

    return pl.pallas_call(
        _fwd_kernel,
        grid=(B,),
        in_specs=[
            pl.BlockSpec((1, S, H), lambda b: (b, 0, 0)),
            whole(ln1_g), whole(ln1_b), whole(in_w), whole(in_b),
            whole(out_w), whole(out_b), whole(ln2_g), whole(ln2_b),
            whole(gate_w), whole(gup), whole(down), whole(head_w),
            whole(head_b2),
        ],
        out_specs=pl.BlockSpec((1, S, VOCAB), lambda b: (b, 0, 0)),
        out_shape=jax.ShapeDtypeStruct((B, S, VOCAB), jnp.float32),
        scratch_shapes=[pltpu.VMEM((Q_CHUNK, H), jnp.float32)],
    )(x, ln1_g, ln1_b, in_w, in_b, out_w, out_b, ln2_g, ln2_b, gate_w, gup,
      down, head_w, head_b2)
